# Initial kernel scaffold; baseline (speedup 1.0000x reference)
import jax
import jax.numpy as jnp
from jax.experimental import pallas as pl


def kernel(neighbors, emb_features, W):
    raise NotImplementedError("write your pallas kernel here")


# speedup vs baseline: 1.3644x; 1.3644x over previous
import functools

import jax
import jax.numpy as jnp
from jax import lax
from jax.experimental import pallas as pl
from jax.experimental.pallas import tpu as pltpu
from jax.experimental.pallas import tpu_sc as plsc

N_NODES = 10000
DEG = 32
D = 128
NC, NS = 2, 16
NW = NC * NS
CH = 16
IB = 128
NB = (CH * DEG) // IB
N_PAD = 10240
NPW = N_PAD // NW
CHUNKS = NPW // CH


def _sc_gather_sum(nbr_flat, seg_ids, emb):
    mesh = plsc.VectorSubcoreMesh(core_axis_name="c", subcore_axis_name="s")

    @functools.partial(
        pl.kernel,
        out_type=jax.ShapeDtypeStruct((N_PAD, D), jnp.float32),
        mesh=mesh,
        scratch_types=[
            pltpu.VMEM((CH * DEG,), jnp.int32),
            pltpu.VMEM((NB, IB), jnp.int32),
            pltpu.VMEM((CH * DEG, D), jnp.float32),
            pltpu.VMEM((CH, D), jnp.float32),
            pltpu.VMEM_SHARED((NS * CH, D), jnp.float32),
            pltpu.SemaphoreType.DMA,
        ],
    )
    def k(nbr_hbm, seg_hbm, emb_hbm, out_hbm, idx_v, seg_v, rows_v, zero_v,
          acc_sh, sem):
        sid = lax.axis_index("s")
        wid = sid * NC + lax.axis_index("c")
        base_node = wid * NPW
        acc_v = acc_sh.at[pl.ds(sid * CH, CH)]

        pltpu.sync_copy(seg_hbm.at[sid], seg_v)

        @pl.loop(0, CH)
        def _(i):
            @pl.loop(0, D, step=16)
            def _(c):
                zero_v[i, pl.ds(c, 16)] = jnp.zeros((16,), jnp.float32)

        @pl.loop(0, CHUNKS)
        def _(g):
            node0 = base_node + g * CH
            pltpu.sync_copy(
                nbr_hbm.at[pl.ds(node0 * DEG, CH * DEG)], idx_v
            )
            for j in range(NB):
                pltpu.async_copy(
                    emb_hbm.at[idx_v.at[pl.ds(j * IB, IB)]],
                    rows_v.at[pl.ds(j * IB, IB)],
                    sem,
                )
            pltpu.sync_copy(zero_v, acc_v)
            for j in range(NB):
                pltpu.make_async_copy(
                    emb_hbm.at[idx_v.at[pl.ds(j * IB, IB)]],
                    rows_v.at[pl.ds(j * IB, IB)],
                    sem,
                ).wait()
            for j in range(NB):
                pltpu.sync_copy(
                    rows_v.at[pl.ds(j * IB, IB)],
                    acc_sh.at[seg_v.at[j]],
                    add=True,
                )
            pltpu.sync_copy(acc_v, out_hbm.at[pl.ds(node0, CH)])

    return k(nbr_flat, seg_ids, emb)


def _tc_post(sums, W):
    BLK = 2048

    def body(x_ref, w_ref, o_ref):
        y = lax.dot_general(
            x_ref[...], w_ref[...],
            (((1,), (1,)), ((), ())),
            preferred_element_type=jnp.float32,
        )
        y = jnp.maximum(y, 0.0)
        norm = jnp.sqrt(jnp.sum(y * y, axis=1, keepdims=True))
        o_ref[...] = y / jnp.maximum(norm, 1e-12)

    return pl.pallas_call(
        body,
        grid=(N_PAD // BLK,),
        in_specs=[
            pl.BlockSpec((BLK, D), lambda i: (i, 0)),
            pl.BlockSpec((D, D), lambda i: (0, 0)),
        ],
        out_specs=pl.BlockSpec((BLK, D), lambda i: (i, 0)),
        out_shape=jax.ShapeDtypeStruct((N_PAD, D), jnp.float32),
    )(sums, W)


def kernel(neighbors, emb_features, W):
    nbr = jnp.pad(neighbors, ((0, N_PAD - N_NODES), (0, 0)))
    nbr_flat = nbr.reshape(-1)
    seg = jnp.repeat(jnp.arange(CH, dtype=jnp.int32), DEG).reshape(NB, IB)
    seg_all = (
        seg[None] + CH * jnp.arange(NS, dtype=jnp.int32)[:, None, None]
    )
    sums = _sc_gather_sum(nbr_flat, seg_all, emb_features)
    out = _tc_post(sums, W)
    return out[:N_NODES]


# speedup vs baseline: 1.5092x; 1.1061x over previous
import functools

import jax
import jax.numpy as jnp
from jax import lax
from jax.experimental import pallas as pl
from jax.experimental.pallas import tpu as pltpu
from jax.experimental.pallas import tpu_sc as plsc

N_NODES = 10000
DEG = 32
D = 128
NC, NS = 2, 16
NW = NC * NS
CH = 8
IB = 128
NB = (CH * DEG) // IB
N_PAD = 10240
NPW = N_PAD // NW
CHUNKS = NPW // CH
PAIRS = CHUNKS // 2
ZR = 32


def _sc_gather_sum(nbr_flat, seg_tab_hbm, emb):
    mesh = plsc.VectorSubcoreMesh(core_axis_name="c", subcore_axis_name="s")

    @functools.partial(
        pl.kernel,
        out_type=jax.ShapeDtypeStruct((N_PAD, D), jnp.float32),
        mesh=mesh,
        scratch_types=[
            pltpu.VMEM((CH * DEG,), jnp.int32),
            pltpu.VMEM((CH * DEG,), jnp.int32),
            pltpu.VMEM((CH * DEG, D), jnp.float32),
            pltpu.VMEM((CH * DEG, D), jnp.float32),
            pltpu.VMEM((CHUNKS * NB, IB), jnp.int32),
            pltpu.VMEM((ZR, D), jnp.float32),
            pltpu.VMEM_SHARED((NS * NPW, D), jnp.float32),
            pltpu.SemaphoreType.DMA,
            pltpu.SemaphoreType.DMA,
        ],
    )
    def k(nbr_hbm, seg_hbm, emb_hbm, out_hbm, idx0, idx1, rows0, rows1,
          seg_tab, zero_v, acc_sh, semG0, semG1):
        sid = lax.axis_index("s")
        wid = sid * NC + lax.axis_index("c")
        base = wid * NPW
        acc0 = sid * NPW

        @pl.loop(0, ZR)
        def _(i):
            @pl.loop(0, D, step=16)
            def _(c):
                zero_v[i, pl.ds(c, 16)] = jnp.zeros((16,), jnp.float32)

        @pl.loop(0, NPW, step=ZR)
        def _(z):
            pltpu.sync_copy(zero_v, acc_sh.at[pl.ds(acc0 + z, ZR)])

        pltpu.sync_copy(seg_hbm.at[sid], seg_tab)

        def load_idx(g, idx_v):
            pltpu.sync_copy(
                nbr_hbm.at[pl.ds((base + g * CH) * DEG, CH * DEG)], idx_v
            )

        def fire(idx_v, rows_v, sem):
            for j in range(NB):
                pltpu.async_copy(
                    emb_hbm.at[idx_v.at[pl.ds(j * IB, IB)]],
                    rows_v.at[pl.ds(j * IB, IB)],
                    sem,
                )

        def drain(idx_v, rows_v, sem):
            for j in range(NB):
                pltpu.make_async_copy(
                    emb_hbm.at[idx_v.at[pl.ds(j * IB, IB)]],
                    rows_v.at[pl.ds(j * IB, IB)],
                    sem,
                ).wait()

        def scatter(g, rows_v):
            for j in range(NB):
                pltpu.sync_copy(
                    rows_v.at[pl.ds(j * IB, IB)],
                    acc_sh.at[seg_tab.at[g * NB + j]],
                    add=True,
                )

        load_idx(0, idx0)
        fire(idx0, rows0, semG0)

        @pl.loop(0, PAIRS)
        def _(t):
            g0 = 2 * t
            load_idx(g0 + 1, idx1)
            fire(idx1, rows1, semG1)
            drain(idx0, rows0, semG0)
            scatter(g0, rows0)

            @pl.when(t < PAIRS - 1)
            def _():
                load_idx(g0 + 2, idx0)
                fire(idx0, rows0, semG0)

            drain(idx1, rows1, semG1)
            scatter(g0 + 1, rows1)

        pltpu.sync_copy(
            acc_sh.at[pl.ds(acc0, NPW)], out_hbm.at[pl.ds(base, NPW)]
        )

    return k(nbr_flat, seg_tab_hbm, emb)


def _tc_post(sums, W):
    BLK = 2048

    def body(x_ref, w_ref, o_ref):
        y = lax.dot_general(
            x_ref[...], w_ref[...],
            (((1,), (1,)), ((), ())),
            preferred_element_type=jnp.float32,
        )
        y = jnp.maximum(y, 0.0)
        norm = jnp.sqrt(jnp.sum(y * y, axis=1, keepdims=True))
        o_ref[...] = y / jnp.maximum(norm, 1e-12)

    return pl.pallas_call(
        body,
        grid=(N_PAD // BLK,),
        in_specs=[
            pl.BlockSpec((BLK, D), lambda i: (i, 0)),
            pl.BlockSpec((D, D), lambda i: (0, 0)),
        ],
        out_specs=pl.BlockSpec((BLK, D), lambda i: (i, 0)),
        out_shape=jax.ShapeDtypeStruct((N_PAD, D), jnp.float32),
    )(sums, W)


def kernel(neighbors, emb_features, W):
    nbr = jnp.pad(neighbors, ((0, N_PAD - N_NODES), (0, 0)))
    nbr_flat = nbr.reshape(-1)
    local_node = jnp.arange(NPW * DEG, dtype=jnp.int32) // DEG
    seg_tab = (
        local_node[None, :]
        + NPW * jnp.arange(NS, dtype=jnp.int32)[:, None]
    ).reshape(NS, CHUNKS * NB, IB)
    sums = _sc_gather_sum(nbr_flat, seg_tab, emb_features)
    out = _tc_post(sums, W)
    return out[:N_NODES]


# speedup vs baseline: 5.0947x; 3.3757x over previous
import functools

import jax
import jax.numpy as jnp
from jax import lax
from jax.experimental import pallas as pl
from jax.experimental.pallas import tpu as pltpu
from jax.experimental.pallas import tpu_sc as plsc

N_NODES = 10000
DEG = 32
D = 128
NC, NS = 2, 16
NW = NC * NS
CH = 8
IB = 128
NB = (CH * DEG) // IB
N_PAD = 10240
NPW = N_PAD // NW
CHUNKS = NPW // CH
PAIRS = CHUNKS // 2
ZR = 32


def _sc_gather_sum(nbr_flat, seg_tab_hbm, emb):
    mesh = plsc.VectorSubcoreMesh(core_axis_name="c", subcore_axis_name="s")

    @functools.partial(
        pl.kernel,
        out_type=jax.ShapeDtypeStruct((N_PAD, D), jnp.float32),
        mesh=mesh,
        scratch_types=[
            pltpu.VMEM((CH * DEG,), jnp.int32),
            pltpu.VMEM((CH * DEG,), jnp.int32),
            pltpu.VMEM((CH * DEG, D), jnp.float32),
            pltpu.VMEM((CH * DEG, D), jnp.float32),
            pltpu.VMEM((CHUNKS * NB, IB), jnp.int32),
            pltpu.VMEM((ZR, D), jnp.float32),
            pltpu.VMEM_SHARED((NS * NPW, D), jnp.float32),
            pltpu.SemaphoreType.DMA,
            pltpu.SemaphoreType.DMA,
        ],
    )
    def k(nbr_hbm, seg_hbm, emb_hbm, out_hbm, idx0, idx1, rows0, rows1,
          seg_tab, zero_v, acc_sh, semG0, semG1):
        sid = lax.axis_index("s")
        wid = sid * NC + lax.axis_index("c")
        base = wid * NPW
        acc0 = sid * NPW

        @pl.loop(0, ZR)
        def _(i):
            @pl.loop(0, D, step=16)
            def _(c):
                zero_v[i, pl.ds(c, 16)] = jnp.zeros((16,), jnp.float32)

        @pl.loop(0, NPW, step=ZR)
        def _(z):
            pltpu.sync_copy(zero_v, acc_sh.at[pl.ds(acc0 + z, ZR)])

        pltpu.sync_copy(seg_hbm.at[sid], seg_tab)

        def load_idx(g, idx_v):
            pltpu.sync_copy(
                nbr_hbm.at[pl.ds((base + g * CH) * DEG, CH * DEG)], idx_v
            )

        def fire(idx_v, rows_v, sem):
            for j in range(NB):
                pltpu.async_copy(
                    emb_hbm.at[idx_v.at[pl.ds(j * IB, IB)]],
                    rows_v.at[pl.ds(j * IB, IB)],
                    sem,
                )

        def drain(idx_v, rows_v, sem):
            for j in range(NB):
                pltpu.make_async_copy(
                    emb_hbm.at[idx_v.at[pl.ds(j * IB, IB)]],
                    rows_v.at[pl.ds(j * IB, IB)],
                    sem,
                ).wait()

        def scatter(g, rows_v):
            for j in range(NB):
                pltpu.sync_copy(
                    rows_v.at[pl.ds(j * IB, IB)],
                    acc_sh.at[seg_tab.at[g * NB + j]],
                    add=True,
                )

        load_idx(0, idx0)
        fire(idx0, rows0, semG0)

        @pl.loop(0, PAIRS)
        def _(t):
            g0 = 2 * t
            load_idx(g0 + 1, idx1)
            fire(idx1, rows1, semG1)
            drain(idx0, rows0, semG0)
            scatter(g0, rows0)

            @pl.when(t < PAIRS - 1)
            def _():
                load_idx(g0 + 2, idx0)
                fire(idx0, rows0, semG0)

            drain(idx1, rows1, semG1)
            scatter(g0 + 1, rows1)

        pltpu.sync_copy(
            acc_sh.at[pl.ds(acc0, NPW)], out_hbm.at[pl.ds(base, NPW)]
        )

    return k(nbr_flat, seg_tab_hbm, emb)


def _tc_post(sums, W):
    BLK = 2048

    def body(x_ref, w_ref, o_ref):
        y = lax.dot_general(
            x_ref[...], w_ref[...],
            (((1,), (1,)), ((), ())),
            preferred_element_type=jnp.float32,
        )
        y = jnp.maximum(y, 0.0)
        norm = jnp.sqrt(jnp.sum(y * y, axis=1, keepdims=True))
        o_ref[...] = y / jnp.maximum(norm, 1e-12)

    return pl.pallas_call(
        body,
        grid=(N_PAD // BLK,),
        in_specs=[
            pl.BlockSpec((BLK, D), lambda i: (i, 0)),
            pl.BlockSpec((D, D), lambda i: (0, 0)),
        ],
        out_specs=pl.BlockSpec((BLK, D), lambda i: (i, 0)),
        out_shape=jax.ShapeDtypeStruct((N_PAD, D), jnp.float32),
    )(sums, W)


def kernel(neighbors, emb_features, W):
    pad_idx = (
        jnp.arange((N_PAD - N_NODES) * DEG, dtype=jnp.int32) % N_NODES
    ).reshape(N_PAD - N_NODES, DEG)
    nbr_flat = jnp.concatenate([neighbors, pad_idx], axis=0).reshape(-1)
    local_node = jnp.arange(NPW * DEG, dtype=jnp.int32) // DEG
    seg_tab = (
        local_node[None, :]
        + NPW * jnp.arange(NS, dtype=jnp.int32)[:, None]
    ).reshape(NS, CHUNKS * NB, IB)
    sums = _sc_gather_sum(nbr_flat, seg_tab, emb_features)
    out = _tc_post(sums, W)
    return out[:N_NODES]


# speedup vs baseline: 5.7251x; 1.1237x over previous
import functools

import jax
import jax.numpy as jnp
from jax import lax
from jax.experimental import pallas as pl
from jax.experimental.pallas import tpu as pltpu
from jax.experimental.pallas import tpu_sc as plsc

N_NODES = 10000
DEG = 32
D = 128
NC, NS = 2, 16
NW = NC * NS
CH = 4
IB = CH * DEG
N_PAD = 10240
NPW = N_PAD // NW
CHUNKS = NPW // CH
R = 4
ZR = 8


def _sc_gather_sum(nbr_flat, seg_tab_hbm, emb):
    mesh = plsc.VectorSubcoreMesh(core_axis_name="c", subcore_axis_name="s")

    @functools.partial(
        pl.kernel,
        out_type=jax.ShapeDtypeStruct((N_PAD, D), jnp.float32),
        mesh=mesh,
        scratch_types=(
            [pltpu.VMEM((CHUNKS, IB), jnp.int32)]
            + [pltpu.VMEM((IB,), jnp.int32) for _ in range(R)]
            + [pltpu.VMEM((IB, D), jnp.float32) for _ in range(R)]
            + [pltpu.VMEM((ZR, D), jnp.float32)]
            + [pltpu.VMEM_SHARED((NS * NPW, D), jnp.float32)]
            + [pltpu.SemaphoreType.DMA for _ in range(R)]
            + [pltpu.SemaphoreType.DMA for _ in range(R)]
        ),
    )
    def k(nbr_hbm, seg_hbm, emb_hbm, out_hbm, seg_tab, *rest):
        idx = rest[:R]
        rows = rest[R:2 * R]
        zero_v = rest[2 * R]
        acc_sh = rest[2 * R + 1]
        semG = rest[2 * R + 2:3 * R + 2]
        semS = rest[3 * R + 2:]

        sid = lax.axis_index("s")
        wid = sid * NC + lax.axis_index("c")
        base = wid * NPW
        acc0 = sid * NPW

        pltpu.sync_copy(seg_hbm.at[sid], seg_tab)

        @pl.loop(0, ZR)
        def _(i):
            @pl.loop(0, D, step=16)
            def _(c):
                zero_v[i, pl.ds(c, 16)] = jnp.zeros((16,), jnp.float32)

        @pl.loop(0, NPW, step=ZR)
        def _(z):
            pltpu.sync_copy(zero_v, acc_sh.at[pl.ds(acc0 + z, ZR)])

        def fire_gather(g, s):
            pltpu.sync_copy(nbr_hbm.at[pl.ds((base * DEG) + g * IB, IB)],
                            idx[s])
            pltpu.async_copy(emb_hbm.at[idx[s]], rows[s], semG[s])

        def drain_gather(g, s):
            pltpu.make_async_copy(
                emb_hbm.at[idx[s]], rows[s], semG[s]
            ).wait()

        def fire_scatter(g, s):
            pltpu.async_copy(
                rows[s], acc_sh.at[seg_tab.at[g]], semS[s], add=True
            )

        def drain_scatter(g, s):
            pltpu.make_async_copy(
                rows[s], acc_sh.at[seg_tab.at[g]], semS[s]
            ).wait()

        for s in range(R - 1):
            fire_gather(s, s)

        @pl.loop(0, CHUNKS, step=R)
        def _(t):
            for s in range(R):
                g = t + s
                drain_gather(g, s)
                fire_scatter(g, s)
                nxt = g + R - 1
                s_nxt = (s + R - 1) % R

                @pl.when(nxt < CHUNKS)
                def _():
                    @pl.when(nxt >= R)
                    def _():
                        drain_scatter(nxt - R, s_nxt)

                    fire_gather(nxt, s_nxt)

        for s in range(R):
            drain_scatter(CHUNKS - R + s, s)

        pltpu.sync_copy(
            acc_sh.at[pl.ds(acc0, NPW)], out_hbm.at[pl.ds(base, NPW)]
        )

    return k(nbr_flat, seg_tab_hbm, emb)


def _tc_post(sums, W):
    BLK = 2000

    def body(x_ref, w_ref, o_ref):
        y = lax.dot_general(
            x_ref[...], w_ref[...],
            (((1,), (1,)), ((), ())),
            preferred_element_type=jnp.float32,
        )
        y = jnp.maximum(y, 0.0)
        norm = jnp.sqrt(jnp.sum(y * y, axis=1, keepdims=True))
        o_ref[...] = y / jnp.maximum(norm, 1e-12)

    return pl.pallas_call(
        body,
        grid=(N_NODES // BLK,),
        in_specs=[
            pl.BlockSpec((BLK, D), lambda i: (i, 0)),
            pl.BlockSpec((D, D), lambda i: (0, 0)),
        ],
        out_specs=pl.BlockSpec((BLK, D), lambda i: (i, 0)),
        out_shape=jax.ShapeDtypeStruct((N_NODES, D), jnp.float32),
    )(sums, W)


def kernel(neighbors, emb_features, W):
    pad_idx = (
        jnp.arange((N_PAD - N_NODES) * DEG, dtype=jnp.int32) % N_NODES
    ).reshape(N_PAD - N_NODES, DEG)
    nbr_flat = jnp.concatenate([neighbors, pad_idx], axis=0).reshape(-1)
    local_node = jnp.arange(NPW * DEG, dtype=jnp.int32) // DEG
    seg_tab = (
        local_node[None, :]
        + NPW * jnp.arange(NS, dtype=jnp.int32)[:, None]
    ).reshape(NS, CHUNKS, IB)
    sums = _sc_gather_sum(nbr_flat, seg_tab, emb_features)
    return _tc_post(sums, W)


# speedup vs baseline: 7.6932x; 1.3438x over previous
import functools

import jax
import jax.numpy as jnp
from jax import lax
from jax.experimental import pallas as pl
from jax.experimental.pallas import tpu as pltpu
from jax.experimental.pallas import tpu_sc as plsc

N_NODES = 10000
DEG = 32
D = 128
NC, NS = 2, 16
NW = NC * NS
N_PAD = 10240
NPW = N_PAD // NW
CH = 80
CHUNKS = NPW // CH


def _sc_gather_sum(nbr_t, emb):
    mesh = plsc.VectorSubcoreMesh(core_axis_name="c", subcore_axis_name="s")

    @functools.partial(
        pl.kernel,
        out_type=jax.ShapeDtypeStruct((N_PAD, D), jnp.float32),
        mesh=mesh,
        scratch_types=[
            pltpu.VMEM((DEG * NPW,), jnp.int32),
            pltpu.VMEM((CH, D), jnp.float32),
            pltpu.VMEM((CH, D), jnp.float32),
            pltpu.SemaphoreType.DMA,
            pltpu.SemaphoreType.DMA,
            pltpu.SemaphoreType.DMA,
            pltpu.SemaphoreType.DMA,
        ],
    )
    def k(nbr_hbm, emb_hbm, out_hbm, idx_all, acc0, acc1, semA0, semA1,
          semO0, semO1):
        sid = lax.axis_index("s")
        wid = sid * NC + lax.axis_index("c")
        base = wid * NPW
        acc = (acc0, acc1)
        semA = (semA0, semA1)
        semO = (semO0, semO1)

        pltpu.sync_copy(nbr_hbm.at[pl.ds(wid * DEG * NPW, DEG * NPW)],
                        idx_all)

        def zero(p):
            a = acc[p]

            @pl.loop(0, CH)
            def _(i):
                @pl.loop(0, D, step=16)
                def _(col):
                    a[i, pl.ds(col, 16)] = jnp.zeros((16,), jnp.float32)

        def fire_adds(c, p):
            for d in range(DEG):
                pltpu.async_copy(
                    emb_hbm.at[idx_all.at[pl.ds(d * NPW + c * CH, CH)]],
                    acc[p],
                    semA[p],
                    add=True,
                )

        def drain_adds(c, p):
            for d in range(DEG):
                pltpu.make_async_copy(
                    emb_hbm.at[idx_all.at[pl.ds(d * NPW + c * CH, CH)]],
                    acc[p],
                    semA[p],
                ).wait()

        def fire_out(c, p):
            pltpu.async_copy(
                acc[p], out_hbm.at[pl.ds(base + c * CH, CH)], semO[p]
            )

        def drain_out(c, p):
            pltpu.make_async_copy(
                acc[p], out_hbm.at[pl.ds(base + c * CH, CH)], semO[p]
            ).wait()

        zero(0)
        fire_adds(0, 0)
        for c in range(CHUNKS):
            p = c % 2
            if c + 1 < CHUNKS:
                if c >= 1:
                    drain_out(c - 1, 1 - p)
                zero(1 - p)
                fire_adds(c + 1, 1 - p)
            drain_adds(c, p)
            fire_out(c, p)
        drain_out(CHUNKS - 2, CHUNKS % 2)
        drain_out(CHUNKS - 1, (CHUNKS - 1) % 2)

    return k(nbr_t, emb)


def _tc_post(sums, W):
    BLK = 2000

    def body(x_ref, w_ref, o_ref):
        y = lax.dot_general(
            x_ref[...], w_ref[...],
            (((1,), (1,)), ((), ())),
            preferred_element_type=jnp.float32,
        )
        y = jnp.maximum(y, 0.0)
        norm = jnp.sqrt(jnp.sum(y * y, axis=1, keepdims=True))
        o_ref[...] = y / jnp.maximum(norm, 1e-12)

    return pl.pallas_call(
        body,
        grid=(N_NODES // BLK,),
        in_specs=[
            pl.BlockSpec((BLK, D), lambda i: (i, 0)),
            pl.BlockSpec((D, D), lambda i: (0, 0)),
        ],
        out_specs=pl.BlockSpec((BLK, D), lambda i: (i, 0)),
        out_shape=jax.ShapeDtypeStruct((N_NODES, D), jnp.float32),
    )(sums, W)


def kernel(neighbors, emb_features, W):
    pad_idx = (
        jnp.arange((N_PAD - N_NODES) * DEG, dtype=jnp.int32) % N_NODES
    ).reshape(N_PAD - N_NODES, DEG)
    nbr = jnp.concatenate([neighbors, pad_idx], axis=0)
    nbr_t = (
        nbr.reshape(NW, NPW, DEG).transpose(0, 2, 1).reshape(-1)
    )
    sums = _sc_gather_sum(nbr_t, emb_features)
    return _tc_post(sums, W)


# speedup vs baseline: 7.7238x; 1.0040x over previous
import functools

import jax
import jax.numpy as jnp
from jax import lax
from jax.experimental import pallas as pl
from jax.experimental.pallas import tpu as pltpu
from jax.experimental.pallas import tpu_sc as plsc

N_NODES = 10000
DEG = 32
D = 128
NC, NS = 2, 16
NW = NC * NS
N_PAD = 10240
NPW = N_PAD // NW
CH = 80
CHUNKS = NPW // CH


def _sc_gather_sum(nbr_t, emb):
    mesh = plsc.VectorSubcoreMesh(core_axis_name="c", subcore_axis_name="s")

    @functools.partial(
        pl.kernel,
        out_type=jax.ShapeDtypeStruct((N_PAD, D), jnp.float32),
        mesh=mesh,
        scratch_types=[
            pltpu.VMEM((DEG * NPW,), jnp.int32),
            pltpu.VMEM((CH, D), jnp.float32),
            pltpu.VMEM((CH, D), jnp.float32),
            pltpu.SemaphoreType.DMA,
            pltpu.SemaphoreType.DMA,
            pltpu.SemaphoreType.DMA,
            pltpu.SemaphoreType.DMA,
        ],
    )
    def k(nbr_hbm, emb_hbm, out_hbm, idx_all, acc0, acc1, semA0, semA1,
          semO0, semO1):
        sid = lax.axis_index("s")
        wid = sid * NC + lax.axis_index("c")
        base = wid * NPW
        acc = (acc0, acc1)
        semA = (semA0, semA1)
        semO = (semO0, semO1)

        idx_cp = pltpu.async_copy(
            nbr_hbm.at[pl.ds(wid * DEG * NPW, DEG * NPW)], idx_all, semO0
        )

        def zero(p):
            a = acc[p]

            @pl.loop(0, CH)
            def _(i):
                @pl.loop(0, D, step=16)
                def _(col):
                    a[i, pl.ds(col, 16)] = jnp.zeros((16,), jnp.float32)

        def fire_adds(c, p):
            for d in range(DEG):
                pltpu.async_copy(
                    emb_hbm.at[idx_all.at[pl.ds(d * NPW + c * CH, CH)]],
                    acc[p],
                    semA[p],
                    add=True,
                )

        def drain_adds(c, p):
            for d in range(DEG):
                pltpu.make_async_copy(
                    emb_hbm.at[idx_all.at[pl.ds(d * NPW + c * CH, CH)]],
                    acc[p],
                    semA[p],
                ).wait()

        def fire_out(c, p):
            pltpu.async_copy(
                acc[p], out_hbm.at[pl.ds(base + c * CH, CH)], semO[p]
            )

        def drain_out(c, p):
            pltpu.make_async_copy(
                acc[p], out_hbm.at[pl.ds(base + c * CH, CH)], semO[p]
            ).wait()

        zero(0)
        zero(1)
        idx_cp.wait()
        fire_adds(0, 0)
        for c in range(CHUNKS):
            p = c % 2
            if c + 1 < CHUNKS:
                if c >= 1:
                    drain_out(c - 1, 1 - p)
                    zero(1 - p)
                fire_adds(c + 1, 1 - p)
            drain_adds(c, p)
            fire_out(c, p)
        drain_out(CHUNKS - 2, CHUNKS % 2)
        drain_out(CHUNKS - 1, (CHUNKS - 1) % 2)

    return k(nbr_t, emb)


def _tc_post(sums, W):
    BLK = 2000

    def body(x_ref, w_ref, o_ref):
        y = lax.dot_general(
            x_ref[...], w_ref[...],
            (((1,), (1,)), ((), ())),
            preferred_element_type=jnp.float32,
        )
        y = jnp.maximum(y, 0.0)
        norm = jnp.sqrt(jnp.sum(y * y, axis=1, keepdims=True))
        o_ref[...] = y / jnp.maximum(norm, 1e-12)

    return pl.pallas_call(
        body,
        grid=(N_NODES // BLK,),
        in_specs=[
            pl.BlockSpec((BLK, D), lambda i: (i, 0)),
            pl.BlockSpec((D, D), lambda i: (0, 0)),
        ],
        out_specs=pl.BlockSpec((BLK, D), lambda i: (i, 0)),
        out_shape=jax.ShapeDtypeStruct((N_NODES, D), jnp.float32),
    )(sums, W)


def kernel(neighbors, emb_features, W):
    pad_idx = (
        jnp.arange((N_PAD - N_NODES) * DEG, dtype=jnp.int32) % N_NODES
    ).reshape(N_PAD - N_NODES, DEG)
    nbr = jnp.concatenate([neighbors, pad_idx], axis=0)
    nbr_t = (
        nbr.reshape(NW, NPW, DEG).transpose(0, 2, 1).reshape(-1)
    )
    sums = _sc_gather_sum(nbr_t, emb_features)
    return _tc_post(sums, W)
